# baseline (speedup 1.0000x reference)
import jax
import jax.numpy as jnp
from jax import lax
from jax.experimental import pallas as pl
from jax.experimental.pallas import tpu as pltpu
from jax.experimental.pallas import tpu_sc as plsc

N = 10000
E = 320000
D = 128
NC = 2
NS = 16
NW = NC * NS
EPT = E // NW
B = 80
NB = EPT // B
NG = 5
CB = NB // NG
RPT = 624
REM = N - RPT * NS
ZR = 48

_mesh = plsc.VectorSubcoreMesh(
    core_axis_name="c", subcore_axis_name="s", num_cores=NC, num_subcores=NS
)


def _sc_body(with_cnt):
    def body(src_hbm, dst_hbm, x_hbm, agg_out, *rest):
        if with_cnt:
            (cnt_out, src_v, dst_v, rows_a, rows_b, rows_c, cnt_v, agg_sh,
             gsem_a, gsem_b, gsem_c, ssem_a, ssem_b, ssem_c) = rest
        else:
            cnt_v = None
            (src_v, dst_v, rows_a, rows_b, rows_c, agg_sh,
             gsem_a, gsem_b, gsem_c, ssem_a, ssem_b, ssem_c) = rest
        c = lax.axis_index("c")
        s = lax.axis_index("s")
        wid = c * NS + s

        zeros16 = jnp.zeros((16,), jnp.float32)

        def zrow(r, carry):
            for j in range(D // 16):
                rows_a[r, pl.ds(j * 16, 16)] = zeros16
            return carry

        lax.fori_loop(0, ZR, zrow, 0)

        if with_cnt:
            def zcnt(i, carry):
                cnt_v[pl.ds(i * 16, 16)] = zeros16
                return carry

            lax.fori_loop(0, N // 16, zcnt, 0)

        zsrc = rows_a.at[pl.ds(0, ZR)]
        for k in range(RPT // ZR):
            pltpu.sync_copy(zsrc, agg_sh.at[pl.ds(s * RPT + k * ZR, ZR)])

        @pl.when(s == NS - 1)
        def _zero_rem():
            pltpu.sync_copy(
                rows_a.at[pl.ds(0, REM)], agg_sh.at[pl.ds(RPT * NS, REM)]
            )

        plsc.subcore_barrier()

        ones16 = jnp.ones((16,), jnp.float32)
        rows = (rows_a, rows_b, rows_c)
        gsems = (gsem_a, gsem_b, gsem_c)
        ssems = (ssem_a, ssem_b, ssem_c)

        def start_g(b, k):
            pltpu.async_copy(x_hbm.at[src_v.at[b]], rows[k], gsems[k])

        def wait_g(b, k):
            pltpu.make_async_copy(
                x_hbm.at[src_v.at[b]], rows[k], gsems[k]
            ).wait()

        def start_s(b, k):
            pltpu.async_copy(
                rows[k], agg_sh.at[dst_v.at[b]], ssems[k], add=True
            )

        def wait_s(b, k):
            pltpu.make_async_copy(
                rows[k], agg_sh.at[dst_v.at[b]], ssems[k]
            ).wait()

        def cnt_add(b):
            if with_cnt:
                for j in range(B // 16):
                    d16 = dst_v[b, pl.ds(j * 16, 16)]
                    plsc.addupdate_scatter(cnt_v, [d16], ones16)

        def chunk(g, carry):
            pltpu.sync_copy(src_hbm.at[wid, g], src_v)
            pltpu.sync_copy(dst_hbm.at[wid, g], dst_v)
            start_g(0, 0)
            start_g(1, 1)

            def group(q, carry2):
                for r in range(3):
                    i = 3 * q + r
                    wait_g(i, r)
                    start_s(i, r)
                    cnt_add(i)
                    prev = (r + 2) % 3
                    if r == 0:
                        @pl.when(q > 0)
                        def _ws():
                            wait_s(3 * q - 1, prev)
                    else:
                        wait_s(i - 1, prev)

                    @pl.when(i + 2 < CB)
                    def _sg():
                        start_g(i + 2, prev)
                return carry2

            lax.fori_loop(0, (CB - 1) // 3, group, 0)
            last = CB - 1
            wait_g(last, 0)
            start_s(last, 0)
            cnt_add(last)
            wait_s(last - 1, 2)
            wait_s(last, 0)
            return carry

        lax.fori_loop(0, NG, chunk, 0)
        plsc.subcore_barrier()

        pltpu.sync_copy(
            agg_sh.at[pl.ds(s * RPT, RPT)], agg_out.at[c, pl.ds(s * RPT, RPT)]
        )

        @pl.when(s == NS - 1)
        def _copy_rem():
            pltpu.sync_copy(
                agg_sh.at[pl.ds(RPT * NS, REM)],
                agg_out.at[c, pl.ds(RPT * NS, REM)],
            )

        if with_cnt:
            pltpu.sync_copy(cnt_v, cnt_out.at[pl.ds(wid * N, N)])

    return body


def _make_sc(with_cnt):
    out_type = [jax.ShapeDtypeStruct((NC, N, D), jnp.float32)]
    if with_cnt:
        out_type.append(jax.ShapeDtypeStruct((NW * N,), jnp.float32))
    return pl.kernel(
        _sc_body(with_cnt),
        out_type=tuple(out_type),
        mesh=_mesh,
        scratch_types=(
            [
                pltpu.VMEM((CB, B), jnp.int32),
                pltpu.VMEM((CB, B), jnp.int32),
                pltpu.VMEM((B, D), jnp.float32),
                pltpu.VMEM((B, D), jnp.float32),
                pltpu.VMEM((B, D), jnp.float32),
            ]
            + ([pltpu.VMEM((N,), jnp.float32)] if with_cnt else [])
            + [
                pltpu.VMEM_SHARED((N, D), jnp.float32),
            ]
            + [pltpu.SemaphoreType.DMA] * 6
        ),
        name="sc_sage_agg_cnt" if with_cnt else "sc_sage_agg",
        compiler_params=pltpu.CompilerParams(needs_layout_passes=False),
    )


_sc_agg_cnt = _make_sc(True)
_sc_agg = _make_sc(False)

R = 2000


def _tc_pre_body(x_ref, wr_ref, b_ref, o_ref):
    o_ref[...] = (
        jnp.dot(x_ref[...], wr_ref[...], preferred_element_type=jnp.float32)
        + b_ref[...]
    )


def _tc_pre(x, wrT, b2d):
    return pl.pallas_call(
        _tc_pre_body,
        grid=(N // R,),
        in_specs=[
            pl.BlockSpec((R, D), lambda i: (i, 0)),
            pl.BlockSpec((D, D), lambda i: (0, 0)),
            pl.BlockSpec((1, D), lambda i: (0, 0)),
        ],
        out_specs=pl.BlockSpec((R, D), lambda i: (i, 0)),
        out_shape=jax.ShapeDtypeStruct((N, D), jnp.float32),
        name="tc_sage_pre",
    )(x, wrT, b2d)


def _tc_post_body(leaky):
    def body(ap_ref, cp_ref, xr_ref, wl_ref, o_ref):
        p = ap_ref[...]
        agg = p[0] + p[1]
        cnt_blk = jnp.sum(cp_ref[...], axis=1)
        scale = 1.0 / jnp.maximum(cnt_blk, 1.0)
        m = agg * scale[:, None]
        y = jnp.dot(m, wl_ref[...], preferred_element_type=jnp.float32)
        y = y + xr_ref[...]
        if leaky:
            y = jnp.where(y > 0, y, 0.01 * y)
        o_ref[...] = y

    return body


def _tc_post(agg_parts, cnt_parts, xr, wlT, leaky):
    return pl.pallas_call(
        _tc_post_body(leaky),
        grid=(N // R,),
        in_specs=[
            pl.BlockSpec((NC, R, D), lambda i: (0, i, 0)),
            pl.BlockSpec((R, NW), lambda i: (i, 0)),
            pl.BlockSpec((R, D), lambda i: (i, 0)),
            pl.BlockSpec((D, D), lambda i: (0, 0)),
        ],
        out_specs=pl.BlockSpec((R, D), lambda i: (i, 0)),
        out_shape=jax.ShapeDtypeStruct((N, D), jnp.float32),
        name="tc_sage_post",
    )(agg_parts, cnt_parts, xr, wlT)


@jax.jit
def kernel(features, edges, edges2, edge_features, additional_feature,
           W1l, W1r, b1, W2l, W2r, b2):
    src = edges[0].astype(jnp.int32).reshape(NW, NG, CB, B)
    dst = edges[1].astype(jnp.int32).reshape(NW, NG, CB, B)

    agg1, cnt_flat = _sc_agg_cnt(src, dst, features)
    cnt_parts = cnt_flat.reshape(NW, N).T
    xr1 = _tc_pre(features, W1r.T, b1.reshape(1, D))
    h = _tc_post(agg1, cnt_parts, xr1, W1l.T, leaky=True)
    (agg2,) = _sc_agg(src, dst, h)
    xr2 = _tc_pre(h, W2r.T, b2.reshape(1, D))
    out = _tc_post(agg2, cnt_parts, xr2, W2l.T, leaky=False)
    return out


# speedup vs baseline: 1.0518x; 1.0518x over previous
import jax
import jax.numpy as jnp
from jax import lax
from jax.experimental import pallas as pl
from jax.experimental.pallas import tpu as pltpu
from jax.experimental.pallas import tpu_sc as plsc

N = 10000
E = 320000
D = 128
NC = 2
NS = 16
NW = NC * NS
EPT = E // NW
B = 80
NB = EPT // B
NG = 5
CB = NB // NG
RPT = 624
REM = N - RPT * NS
ZR = 48

_mesh = plsc.VectorSubcoreMesh(
    core_axis_name="c", subcore_axis_name="s", num_cores=NC, num_subcores=NS
)


def _sc_body(with_cnt):
    def body(edges_hbm, x_hbm, agg_out, *rest):
        if with_cnt:
            (cnt_out, src_v, dst_v, rows_a, rows_b, rows_c, cnt_v, agg_sh,
             gsem_a, gsem_b, gsem_c, ssem_a, ssem_b, ssem_c) = rest
        else:
            cnt_v = None
            (src_v, dst_v, rows_a, rows_b, rows_c, agg_sh,
             gsem_a, gsem_b, gsem_c, ssem_a, ssem_b, ssem_c) = rest
        c = lax.axis_index("c")
        s = lax.axis_index("s")
        wid = c * NS + s

        zeros16 = jnp.zeros((16,), jnp.float32)

        def zrow(r, carry):
            for j in range(D // 16):
                rows_a[r, pl.ds(j * 16, 16)] = zeros16
            return carry

        lax.fori_loop(0, ZR, zrow, 0)

        if with_cnt:
            def zcnt(i, carry):
                cnt_v[pl.ds(i * 16, 16)] = zeros16
                return carry

            lax.fori_loop(0, N // 16, zcnt, 0)

        zsrc = rows_a.at[pl.ds(0, ZR)]
        for k in range(RPT // ZR):
            pltpu.sync_copy(zsrc, agg_sh.at[pl.ds(s * RPT + k * ZR, ZR)])

        @pl.when(s == NS - 1)
        def _zero_rem():
            pltpu.sync_copy(
                rows_a.at[pl.ds(0, REM)], agg_sh.at[pl.ds(RPT * NS, REM)]
            )

        plsc.subcore_barrier()

        ones16 = jnp.ones((16,), jnp.float32)
        rows = (rows_a, rows_b, rows_c)
        gsems = (gsem_a, gsem_b, gsem_c)
        ssems = (ssem_a, ssem_b, ssem_c)

        def start_g(b, k):
            pltpu.async_copy(x_hbm.at[src_v.at[b]], rows[k], gsems[k])

        def wait_g(b, k):
            pltpu.make_async_copy(
                x_hbm.at[src_v.at[b]], rows[k], gsems[k]
            ).wait()

        def start_s(b, k):
            pltpu.async_copy(
                rows[k], agg_sh.at[dst_v.at[b]], ssems[k], add=True
            )

        def wait_s(b, k):
            pltpu.make_async_copy(
                rows[k], agg_sh.at[dst_v.at[b]], ssems[k]
            ).wait()

        def cnt_add(b):
            if with_cnt:
                for j in range(B // 16):
                    d16 = dst_v[b, pl.ds(j * 16, 16)]
                    plsc.addupdate_scatter(cnt_v, [d16], ones16)

        def chunk(g, carry):
            pltpu.sync_copy(edges_hbm.at[0, wid, g], src_v)
            pltpu.sync_copy(edges_hbm.at[1, wid, g], dst_v)
            start_g(0, 0)
            start_g(1, 1)

            def group(q, carry2):
                for r in range(3):
                    i = 3 * q + r
                    wait_g(i, r)
                    start_s(i, r)
                    cnt_add(i)
                    prev = (r + 2) % 3
                    if r == 0:
                        @pl.when(q > 0)
                        def _ws():
                            wait_s(3 * q - 1, prev)
                    else:
                        wait_s(i - 1, prev)

                    @pl.when(i + 2 < CB)
                    def _sg():
                        start_g(i + 2, prev)
                return carry2

            lax.fori_loop(0, (CB - 1) // 3, group, 0)
            last = CB - 1
            wait_g(last, 0)
            start_s(last, 0)
            cnt_add(last)
            wait_s(last - 1, 2)
            wait_s(last, 0)
            return carry

        lax.fori_loop(0, NG, chunk, 0)
        plsc.subcore_barrier()

        pltpu.sync_copy(
            agg_sh.at[pl.ds(s * RPT, RPT)], agg_out.at[c, pl.ds(s * RPT, RPT)]
        )

        @pl.when(s == NS - 1)
        def _copy_rem():
            pltpu.sync_copy(
                agg_sh.at[pl.ds(RPT * NS, REM)],
                agg_out.at[c, pl.ds(RPT * NS, REM)],
            )

        if with_cnt:
            pltpu.sync_copy(cnt_v, cnt_out.at[pl.ds(wid * N, N)])

    return body


def _make_sc(with_cnt):
    out_type = [jax.ShapeDtypeStruct((NC, N, D), jnp.float32)]
    if with_cnt:
        out_type.append(jax.ShapeDtypeStruct((NW * N,), jnp.float32))
    return pl.kernel(
        _sc_body(with_cnt),
        out_type=tuple(out_type),
        mesh=_mesh,
        scratch_types=(
            [
                pltpu.VMEM((CB, B), jnp.int32),
                pltpu.VMEM((CB, B), jnp.int32),
                pltpu.VMEM((B, D), jnp.float32),
                pltpu.VMEM((B, D), jnp.float32),
                pltpu.VMEM((B, D), jnp.float32),
            ]
            + ([pltpu.VMEM((N,), jnp.float32)] if with_cnt else [])
            + [
                pltpu.VMEM_SHARED((N, D), jnp.float32),
            ]
            + [pltpu.SemaphoreType.DMA] * 6
        ),
        name="sc_sage_agg_cnt" if with_cnt else "sc_sage_agg",
        compiler_params=pltpu.CompilerParams(needs_layout_passes=False),
    )


_sc_agg_cnt = _make_sc(True)
_sc_agg = _make_sc(False)

def _tc_pre_body(x_ref, wr_ref, b_ref, o_ref):
    o_ref[...] = (
        jnp.dot(x_ref[...], wr_ref[...], preferred_element_type=jnp.float32)
        + b_ref[...]
    )


def _tc_pre(x, wrT, b2d):
    return pl.pallas_call(
        _tc_pre_body,
        out_shape=jax.ShapeDtypeStruct((N, D), jnp.float32),
        name="tc_sage_pre",
    )(x, wrT, b2d)


def _mean_lin(ap_ref, cp_ref, xr_ref, wl_ref):
    p = ap_ref[...]
    agg = p[0] + p[1]
    cnt = jnp.sum(cp_ref[...], axis=0)
    scale = 1.0 / jnp.maximum(cnt, 1.0)
    m = agg * scale[:, None]
    y = jnp.dot(m, wl_ref[...], preferred_element_type=jnp.float32)
    return y + xr_ref[...]


def _tc_mid_body(ap_ref, cp_ref, xr_ref, wl_ref, wr2_ref, b2_ref, h_ref,
                 xr2_ref):
    y = _mean_lin(ap_ref, cp_ref, xr_ref, wl_ref)
    h = jnp.where(y > 0, y, 0.01 * y)
    h_ref[...] = h
    xr2_ref[...] = (
        jnp.dot(h, wr2_ref[...], preferred_element_type=jnp.float32)
        + b2_ref[...]
    )


def _tc_mid(agg_parts, cnt_flat, xr1, w1lT, w2rT, b2d):
    return pl.pallas_call(
        _tc_mid_body,
        out_shape=(
            jax.ShapeDtypeStruct((N, D), jnp.float32),
            jax.ShapeDtypeStruct((N, D), jnp.float32),
        ),
        name="tc_sage_mid",
    )(agg_parts, cnt_flat, xr1, w1lT, w2rT, b2d)


def _tc_post_body(ap_ref, cp_ref, xr_ref, wl_ref, o_ref):
    o_ref[...] = _mean_lin(ap_ref, cp_ref, xr_ref, wl_ref)


def _tc_post(agg_parts, cnt_flat, xr, wlT):
    return pl.pallas_call(
        _tc_post_body,
        out_shape=jax.ShapeDtypeStruct((N, D), jnp.float32),
        name="tc_sage_post",
    )(agg_parts, cnt_flat, xr, wlT)


@jax.jit
def kernel(features, edges, edges2, edge_features, additional_feature,
           W1l, W1r, b1, W2l, W2r, b2):
    e5 = edges.astype(jnp.int32).reshape(2, NW, NG, CB, B)

    agg1, cnt_flat = _sc_agg_cnt(e5, features)
    cnt2d = cnt_flat.reshape(NW, N)
    xr1 = _tc_pre(features, W1r.T, b1.reshape(1, D))
    h, xr2 = _tc_mid(agg1, cnt2d, xr1, W1l.T, W2r.T, b2.reshape(1, D))
    (agg2,) = _sc_agg(e5, h)
    out = _tc_post(agg2, cnt2d, xr2, W2l.T)
    return out


# speedup vs baseline: 1.0592x; 1.0070x over previous
import jax
import jax.numpy as jnp
from jax import lax
from jax.experimental import pallas as pl
from jax.experimental.pallas import tpu as pltpu
from jax.experimental.pallas import tpu_sc as plsc

N = 10000
E = 320000
D = 128
NC = 2
NS = 16
NW = NC * NS
EPT = E // NW
B = 80
NB = EPT // B
NG = 5
CB = NB // NG
RPT = 624
REM = N - RPT * NS
ZR = 48

_mesh = plsc.VectorSubcoreMesh(
    core_axis_name="c", subcore_axis_name="s", num_cores=NC, num_subcores=NS
)


def _sc_body(with_cnt):
    def body(edges_hbm, x_hbm, agg_out, *rest):
        if with_cnt:
            (cnt_out, src_v, dst_v, rows_a, rows_b, rows_c, cnt_v, agg_sh,
             gsem_a, gsem_b, gsem_c, ssem_a, ssem_b, ssem_c) = rest
        else:
            cnt_v = None
            (src_v, dst_v, rows_a, rows_b, rows_c, agg_sh,
             gsem_a, gsem_b, gsem_c, ssem_a, ssem_b, ssem_c) = rest
        zeros16i = jnp.zeros((16,), jnp.int32)
        c = lax.axis_index("c")
        s = lax.axis_index("s")
        wid = c * NS + s

        zeros16 = jnp.zeros((16,), jnp.float32)

        def zrow(r, carry):
            for j in range(D // 16):
                rows_a[r, pl.ds(j * 16, 16)] = zeros16
            return carry

        lax.fori_loop(0, ZR, zrow, 0)

        if with_cnt:
            def zcnt(i, carry):
                cnt_v[0, pl.ds(i * 16, 16)] = zeros16
                return carry

            lax.fori_loop(0, N // 16, zcnt, 0)

        zsrc = rows_a.at[pl.ds(0, ZR)]
        for k in range(RPT // ZR):
            pltpu.sync_copy(zsrc, agg_sh.at[pl.ds(s * RPT + k * ZR, ZR)])

        @pl.when(s == NS - 1)
        def _zero_rem():
            pltpu.sync_copy(
                rows_a.at[pl.ds(0, REM)], agg_sh.at[pl.ds(RPT * NS, REM)]
            )

        plsc.subcore_barrier()

        ones16 = jnp.ones((16,), jnp.float32)
        rows = (rows_a, rows_b, rows_c)
        gsems = (gsem_a, gsem_b, gsem_c)
        ssems = (ssem_a, ssem_b, ssem_c)

        def start_g(b, k):
            pltpu.async_copy(x_hbm.at[src_v.at[b]], rows[k], gsems[k])

        def wait_g(b, k):
            pltpu.make_async_copy(
                x_hbm.at[src_v.at[b]], rows[k], gsems[k]
            ).wait()

        def start_s(b, k):
            pltpu.async_copy(
                rows[k], agg_sh.at[dst_v.at[b]], ssems[k], add=True
            )

        def wait_s(b, k):
            pltpu.make_async_copy(
                rows[k], agg_sh.at[dst_v.at[b]], ssems[k]
            ).wait()

        def cnt_add(b):
            if with_cnt:
                for j in range(B // 16):
                    d16 = dst_v[b, pl.ds(j * 16, 16)]
                    plsc.addupdate_scatter(cnt_v, [zeros16i, d16], ones16)

        def chunk(g, carry):
            pltpu.sync_copy(edges_hbm.at[0, wid, g], src_v)
            pltpu.sync_copy(edges_hbm.at[1, wid, g], dst_v)
            start_g(0, 0)
            start_g(1, 1)

            def group(q, carry2):
                for r in range(3):
                    i = 3 * q + r
                    wait_g(i, r)
                    start_s(i, r)
                    cnt_add(i)
                    prev = (r + 2) % 3
                    if r == 0:
                        @pl.when(q > 0)
                        def _ws():
                            wait_s(3 * q - 1, prev)
                    else:
                        wait_s(i - 1, prev)

                    @pl.when(i + 2 < CB)
                    def _sg():
                        start_g(i + 2, prev)
                return carry2

            lax.fori_loop(0, (CB - 1) // 3, group, 0)
            last = CB - 1
            wait_g(last, 0)
            start_s(last, 0)
            cnt_add(last)
            wait_s(last - 1, 2)
            wait_s(last, 0)
            return carry

        lax.fori_loop(0, NG, chunk, 0)
        plsc.subcore_barrier()

        pltpu.sync_copy(
            agg_sh.at[pl.ds(s * RPT, RPT)], agg_out.at[c, pl.ds(s * RPT, RPT)]
        )

        @pl.when(s == NS - 1)
        def _copy_rem():
            pltpu.sync_copy(
                agg_sh.at[pl.ds(RPT * NS, REM)],
                agg_out.at[c, pl.ds(RPT * NS, REM)],
            )

        if with_cnt:
            pltpu.sync_copy(cnt_v, cnt_out.at[wid])

    return body


def _make_sc(with_cnt):
    out_type = [jax.ShapeDtypeStruct((NC, N, D), jnp.float32)]
    if with_cnt:
        out_type.append(jax.ShapeDtypeStruct((NW, 1, N), jnp.float32))
    return pl.kernel(
        _sc_body(with_cnt),
        out_type=tuple(out_type),
        mesh=_mesh,
        scratch_types=(
            [
                pltpu.VMEM((CB, B), jnp.int32),
                pltpu.VMEM((CB, B), jnp.int32),
                pltpu.VMEM((B, D), jnp.float32),
                pltpu.VMEM((B, D), jnp.float32),
                pltpu.VMEM((B, D), jnp.float32),
            ]
            + ([pltpu.VMEM((1, N), jnp.float32)] if with_cnt else [])
            + [
                pltpu.VMEM_SHARED((N, D), jnp.float32),
            ]
            + [pltpu.SemaphoreType.DMA] * 6
        ),
        name="sc_sage_agg_cnt" if with_cnt else "sc_sage_agg",
        compiler_params=pltpu.CompilerParams(needs_layout_passes=False),
    )


_sc_agg_cnt = _make_sc(True)
_sc_agg = _make_sc(False)

def _tc_pre_body(x_ref, wr_ref, b_ref, o_ref):
    o_ref[...] = (
        jnp.dot(x_ref[...], wr_ref[...], preferred_element_type=jnp.float32)
        + b_ref[...]
    )


def _tc_pre(x, wrT, b2d):
    return pl.pallas_call(
        _tc_pre_body,
        out_shape=jax.ShapeDtypeStruct((N, D), jnp.float32),
        name="tc_sage_pre",
    )(x, wrT, b2d)


def _mean_lin(ap_ref, cp_ref, xr_ref, wl_ref):
    p = ap_ref[...]
    agg = p[0] + p[1]
    cnt = jnp.sum(cp_ref[...], axis=(0, 1))
    scale = 1.0 / jnp.maximum(cnt, 1.0)
    m = agg * scale[:, None]
    y = jnp.dot(m, wl_ref[...], preferred_element_type=jnp.float32)
    return y + xr_ref[...]


def _tc_mid_body(ap_ref, cp_ref, xr_ref, wl_ref, wr2_ref, b2_ref, h_ref,
                 xr2_ref):
    y = _mean_lin(ap_ref, cp_ref, xr_ref, wl_ref)
    h = jnp.where(y > 0, y, 0.01 * y)
    h_ref[...] = h
    xr2_ref[...] = (
        jnp.dot(h, wr2_ref[...], preferred_element_type=jnp.float32)
        + b2_ref[...]
    )


def _tc_mid(agg_parts, cnt_flat, xr1, w1lT, w2rT, b2d):
    return pl.pallas_call(
        _tc_mid_body,
        out_shape=(
            jax.ShapeDtypeStruct((N, D), jnp.float32),
            jax.ShapeDtypeStruct((N, D), jnp.float32),
        ),
        name="tc_sage_mid",
    )(agg_parts, cnt_flat, xr1, w1lT, w2rT, b2d)


def _tc_post_body(ap_ref, cp_ref, xr_ref, wl_ref, o_ref):
    o_ref[...] = _mean_lin(ap_ref, cp_ref, xr_ref, wl_ref)


def _tc_post(agg_parts, cnt_flat, xr, wlT):
    return pl.pallas_call(
        _tc_post_body,
        out_shape=jax.ShapeDtypeStruct((N, D), jnp.float32),
        name="tc_sage_post",
    )(agg_parts, cnt_flat, xr, wlT)


@jax.jit
def kernel(features, edges, edges2, edge_features, additional_feature,
           W1l, W1r, b1, W2l, W2r, b2):
    e5 = edges.astype(jnp.int32).reshape(2, NW, NG, CB, B)

    agg1, cnt3 = _sc_agg_cnt(e5, features)
    xr1 = _tc_pre(features, W1r.T, b1.reshape(1, D))
    h, xr2 = _tc_mid(agg1, cnt3, xr1, W1l.T, W2r.T, b2.reshape(1, D))
    (agg2,) = _sc_agg(e5, h)
    out = _tc_post(agg2, cnt3, xr2, W2l.T)
    return out


# speedup vs baseline: 1.0946x; 1.0334x over previous
import jax
import jax.numpy as jnp
from jax import lax
from jax.experimental import pallas as pl
from jax.experimental.pallas import tpu as pltpu
from jax.experimental.pallas import tpu_sc as plsc

N = 10000
E = 320000
D = 128
NC = 2
NS = 16
NW = NC * NS
EPT = E // NW
B = 80
NB = EPT // B
NG = 5
CB = NB // NG
RPT = 624
REM = N - RPT * NS
ZR = 48

_mesh = plsc.VectorSubcoreMesh(
    core_axis_name="c", subcore_axis_name="s", num_cores=NC, num_subcores=NS
)


def _sc_body(with_cnt):
    def body(edges_hbm, x_hbm, agg_out, *rest):
        if with_cnt:
            (cnt_out, src_v, dst_v, rows_a, rows_b, rows_c, cnt_v, agg_sh,
             gsem_a, gsem_b, gsem_c, ssem_a, ssem_b, ssem_c) = rest
        else:
            cnt_v = None
            (src_v, dst_v, rows_a, rows_b, rows_c, agg_sh,
             gsem_a, gsem_b, gsem_c, ssem_a, ssem_b, ssem_c) = rest
        zeros16i = jnp.zeros((16,), jnp.int32)
        c = lax.axis_index("c")
        s = lax.axis_index("s")
        wid = c * NS + s

        zeros16 = jnp.zeros((16,), jnp.float32)

        pltpu.async_copy(edges_hbm.at[0, wid, 0], src_v, gsem_a)
        pltpu.async_copy(edges_hbm.at[1, wid, 0], dst_v, gsem_b)

        def zrow(r, carry):
            for j in range(D // 16):
                rows_a[r, pl.ds(j * 16, 16)] = zeros16
            return carry

        lax.fori_loop(0, ZR, zrow, 0)

        zsrc = rows_a.at[pl.ds(0, ZR)]
        for k in range(RPT // ZR):
            pltpu.async_copy(zsrc, agg_sh.at[pl.ds(s * RPT + k * ZR, ZR)],
                             ssem_a)

        @pl.when(s == NS - 1)
        def _zero_rem():
            pltpu.async_copy(
                rows_a.at[pl.ds(0, REM)], agg_sh.at[pl.ds(RPT * NS, REM)],
                ssem_b,
            )

        if with_cnt:
            def zcnt(i, carry):
                cnt_v[0, pl.ds(i * 16, 16)] = zeros16
                return carry

            lax.fori_loop(0, N // 16, zcnt, 0)

        for k in range(RPT // ZR):
            pltpu.make_async_copy(
                zsrc, agg_sh.at[pl.ds(s * RPT + k * ZR, ZR)], ssem_a
            ).wait()

        @pl.when(s == NS - 1)
        def _zero_rem_wait():
            pltpu.make_async_copy(
                rows_a.at[pl.ds(0, REM)], agg_sh.at[pl.ds(RPT * NS, REM)],
                ssem_b,
            ).wait()

        pltpu.make_async_copy(edges_hbm.at[0, wid, 0], src_v, gsem_a).wait()
        pltpu.make_async_copy(edges_hbm.at[1, wid, 0], dst_v, gsem_b).wait()
        plsc.subcore_barrier()

        ones16 = jnp.ones((16,), jnp.float32)
        rows = (rows_a, rows_b, rows_c)
        gsems = (gsem_a, gsem_b, gsem_c)
        ssems = (ssem_a, ssem_b, ssem_c)

        def start_g(b, k):
            pltpu.async_copy(x_hbm.at[src_v.at[b]], rows[k], gsems[k])

        def wait_g(b, k):
            pltpu.make_async_copy(
                x_hbm.at[src_v.at[b]], rows[k], gsems[k]
            ).wait()

        def start_s(b, k):
            pltpu.async_copy(
                rows[k], agg_sh.at[dst_v.at[b]], ssems[k], add=True
            )

        def wait_s(b, k):
            pltpu.make_async_copy(
                rows[k], agg_sh.at[dst_v.at[b]], ssems[k]
            ).wait()

        def cnt_add(b):
            if with_cnt:
                for j in range(B // 16):
                    d16 = dst_v[b, pl.ds(j * 16, 16)]
                    plsc.addupdate_scatter(cnt_v, [zeros16i, d16], ones16)

        def chunk(g, carry):
            @pl.when(g > 0)
            def _load():
                pltpu.sync_copy(edges_hbm.at[0, wid, g], src_v)
                pltpu.sync_copy(edges_hbm.at[1, wid, g], dst_v)

            start_g(0, 0)
            start_g(1, 1)

            def group(q, carry2):
                for r in range(3):
                    i = 3 * q + r
                    wait_g(i, r)
                    start_s(i, r)
                    cnt_add(i)
                    prev = (r + 2) % 3
                    if r == 0:
                        @pl.when(q > 0)
                        def _ws():
                            wait_s(3 * q - 1, prev)
                    else:
                        wait_s(i - 1, prev)

                    @pl.when(i + 2 < CB)
                    def _sg():
                        start_g(i + 2, prev)
                return carry2

            lax.fori_loop(0, (CB - 1) // 3, group, 0)
            last = CB - 1
            wait_g(last, 0)
            start_s(last, 0)
            cnt_add(last)
            wait_s(last - 1, 2)
            wait_s(last, 0)
            return carry

        lax.fori_loop(0, NG, chunk, 0)
        plsc.subcore_barrier()

        pltpu.sync_copy(
            agg_sh.at[pl.ds(s * RPT, RPT)], agg_out.at[c, pl.ds(s * RPT, RPT)]
        )

        @pl.when(s == NS - 1)
        def _copy_rem():
            pltpu.sync_copy(
                agg_sh.at[pl.ds(RPT * NS, REM)],
                agg_out.at[c, pl.ds(RPT * NS, REM)],
            )

        if with_cnt:
            pltpu.sync_copy(cnt_v, cnt_out.at[wid])

    return body


def _make_sc(with_cnt):
    out_type = [jax.ShapeDtypeStruct((NC, N, D), jnp.float32)]
    if with_cnt:
        out_type.append(jax.ShapeDtypeStruct((NW, 1, N), jnp.float32))
    return pl.kernel(
        _sc_body(with_cnt),
        out_type=tuple(out_type),
        mesh=_mesh,
        scratch_types=(
            [
                pltpu.VMEM((CB, B), jnp.int32),
                pltpu.VMEM((CB, B), jnp.int32),
                pltpu.VMEM((B, D), jnp.float32),
                pltpu.VMEM((B, D), jnp.float32),
                pltpu.VMEM((B, D), jnp.float32),
            ]
            + ([pltpu.VMEM((1, N), jnp.float32)] if with_cnt else [])
            + [
                pltpu.VMEM_SHARED((N, D), jnp.float32),
            ]
            + [pltpu.SemaphoreType.DMA] * 6
        ),
        name="sc_sage_agg_cnt" if with_cnt else "sc_sage_agg",
        compiler_params=pltpu.CompilerParams(needs_layout_passes=False),
    )


_sc_agg_cnt = _make_sc(True)
_sc_agg = _make_sc(False)

def _tc_pre_body(x_ref, wr_ref, b_ref, o_ref):
    o_ref[...] = (
        jnp.dot(x_ref[...], wr_ref[...], preferred_element_type=jnp.float32)
        + b_ref[...]
    )


def _tc_pre(x, wrT, b2d):
    return pl.pallas_call(
        _tc_pre_body,
        out_shape=jax.ShapeDtypeStruct((N, D), jnp.float32),
        name="tc_sage_pre",
    )(x, wrT, b2d)


def _mean_lin(ap_ref, cp_ref, xr_ref, wl_ref):
    p = ap_ref[...]
    agg = p[0] + p[1]
    cnt = jnp.sum(cp_ref[...], axis=(0, 1))
    scale = 1.0 / jnp.maximum(cnt, 1.0)
    m = agg * scale[:, None]
    y = jnp.dot(m, wl_ref[...], preferred_element_type=jnp.float32)
    return y + xr_ref[...]


def _tc_post_body(leaky):
    def body(ap_ref, cp_ref, xr_ref, wl_ref, o_ref):
        y = _mean_lin(ap_ref, cp_ref, xr_ref, wl_ref)
        if leaky:
            y = jnp.where(y > 0, y, 0.01 * y)
        o_ref[...] = y

    return body


def _tc_post(agg_parts, cnt_flat, xr, wlT, leaky):
    return pl.pallas_call(
        _tc_post_body(leaky),
        out_shape=jax.ShapeDtypeStruct((N, D), jnp.float32),
        name="tc_sage_post",
    )(agg_parts, cnt_flat, xr, wlT)


@jax.jit
def kernel(features, edges, edges2, edge_features, additional_feature,
           W1l, W1r, b1, W2l, W2r, b2):
    e5 = edges.astype(jnp.int32).reshape(2, NW, NG, CB, B)

    agg1, cnt3 = _sc_agg_cnt(e5, features)
    xr1 = _tc_pre(features, W1r.T, b1.reshape(1, D))
    h = _tc_post(agg1, cnt3, xr1, W1l.T, leaky=True)
    (agg2,) = _sc_agg(e5, h)
    xr2 = _tc_pre(h, W2r.T, b2.reshape(1, D))
    out = _tc_post(agg2, cnt3, xr2, W2l.T, leaky=False)
    return out


# speedup vs baseline: 1.0951x; 1.0005x over previous
import jax
import jax.numpy as jnp
from jax import lax
from jax.experimental import pallas as pl
from jax.experimental.pallas import tpu as pltpu
from jax.experimental.pallas import tpu_sc as plsc

N = 10000
E = 320000
D = 128
NC = 2
NS = 16
NW = NC * NS
EPT = E // NW
B = 80
NB = EPT // B
NG = 5
CB = NB // NG
RPT = 624
REM = N - RPT * NS
ZR = 48

_mesh = plsc.VectorSubcoreMesh(
    core_axis_name="c", subcore_axis_name="s", num_cores=NC, num_subcores=NS
)


def _sc_body(with_cnt):
    def body(edges_hbm, x_hbm, agg_out, *rest):
        if with_cnt:
            (cnt_out, src_v, dst_v, rows_a, rows_b, rows_c, cnt_v, agg_sh,
             gsem_a, gsem_b, gsem_c, ssem_a, ssem_b, ssem_c) = rest
        else:
            cnt_v = None
            (src_v, dst_v, rows_a, rows_b, rows_c, agg_sh,
             gsem_a, gsem_b, gsem_c, ssem_a, ssem_b, ssem_c) = rest
        zeros16i = jnp.zeros((16,), jnp.int32)
        c = lax.axis_index("c")
        s = lax.axis_index("s")
        wid = c * NS + s

        zeros16 = jnp.zeros((16,), jnp.float32)

        pltpu.async_copy(edges_hbm.at[0, wid, 0], src_v, gsem_a)
        pltpu.async_copy(edges_hbm.at[1, wid, 0], dst_v, gsem_b)

        def zrow(r, carry):
            for j in range(D // 16):
                rows_a[r, pl.ds(j * 16, 16)] = zeros16
            return carry

        lax.fori_loop(0, ZR, zrow, 0)

        zsrc = rows_a.at[pl.ds(0, ZR)]
        for k in range(RPT // ZR):
            pltpu.async_copy(zsrc, agg_sh.at[pl.ds(s * RPT + k * ZR, ZR)],
                             ssem_a)

        @pl.when(s == NS - 1)
        def _zero_rem():
            pltpu.async_copy(
                rows_a.at[pl.ds(0, REM)], agg_sh.at[pl.ds(RPT * NS, REM)],
                ssem_b,
            )

        if with_cnt:
            def zcnt(i, carry):
                cnt_v[0, pl.ds(i * 16, 16)] = zeros16
                return carry

            lax.fori_loop(0, N // 16, zcnt, 0)

        for k in range(RPT // ZR):
            pltpu.make_async_copy(
                zsrc, agg_sh.at[pl.ds(s * RPT + k * ZR, ZR)], ssem_a
            ).wait()

        @pl.when(s == NS - 1)
        def _zero_rem_wait():
            pltpu.make_async_copy(
                rows_a.at[pl.ds(0, REM)], agg_sh.at[pl.ds(RPT * NS, REM)],
                ssem_b,
            ).wait()

        pltpu.make_async_copy(edges_hbm.at[0, wid, 0], src_v, gsem_a).wait()
        pltpu.make_async_copy(edges_hbm.at[1, wid, 0], dst_v, gsem_b).wait()
        plsc.subcore_barrier()

        ones16 = jnp.ones((16,), jnp.float32)
        rows = (rows_a, rows_b, rows_c)
        gsems = (gsem_a, gsem_b, gsem_c)
        ssems = (ssem_a, ssem_b, ssem_c)

        def start_g(sv, b, k):
            pltpu.async_copy(x_hbm.at[sv.at[b]], rows[k], gsems[k])

        def wait_g(sv, b, k):
            pltpu.make_async_copy(
                x_hbm.at[sv.at[b]], rows[k], gsems[k]
            ).wait()

        def start_s(dv, b, k):
            pltpu.async_copy(
                rows[k], agg_sh.at[dv.at[b]], ssems[k], add=True
            )

        def wait_s(dv, b, k):
            pltpu.make_async_copy(
                rows[k], agg_sh.at[dv.at[b]], ssems[k]
            ).wait()

        def cnt_add(dv, b):
            if with_cnt:
                for j in range(B // 16):
                    d16 = dv[b, pl.ds(j * 16, 16)]
                    plsc.addupdate_scatter(cnt_v, [zeros16i, d16], ones16)

        def chunk(g, carry):
            sv = src_v
            dv = dst_v

            @pl.when(g > 0)
            def _load():
                pltpu.sync_copy(edges_hbm.at[0, wid, g], src_v)
                pltpu.sync_copy(edges_hbm.at[1, wid, g], dst_v)

            start_g(sv, 0, 0)
            start_g(sv, 1, 1)

            def group(q, carry2):
                for r in range(3):
                    i = 3 * q + r
                    wait_g(sv, i, r)
                    start_s(dv, i, r)
                    cnt_add(dv, i)
                    prev = (r + 2) % 3
                    if r == 0:
                        @pl.when(q > 0)
                        def _ws():
                            wait_s(dv, 3 * q - 1, prev)
                    else:
                        wait_s(dv, i - 1, prev)

                    @pl.when(i + 2 < CB)
                    def _sg():
                        start_g(sv, i + 2, prev)
                return carry2

            lax.fori_loop(0, (CB - 1) // 3, group, 0)
            last = CB - 1
            wait_g(sv, last, 0)
            start_s(dv, last, 0)
            cnt_add(dv, last)
            wait_s(dv, last - 1, 2)
            wait_s(dv, last, 0)
            return carry

        lax.fori_loop(0, NG, chunk, 0)
        plsc.subcore_barrier()

        pltpu.sync_copy(
            agg_sh.at[pl.ds(s * RPT, RPT)], agg_out.at[c, pl.ds(s * RPT, RPT)]
        )

        @pl.when(s == NS - 1)
        def _copy_rem():
            pltpu.sync_copy(
                agg_sh.at[pl.ds(RPT * NS, REM)],
                agg_out.at[c, pl.ds(RPT * NS, REM)],
            )

        if with_cnt:
            pltpu.sync_copy(cnt_v, cnt_out.at[wid])

    return body


def _make_sc(with_cnt):
    out_type = [jax.ShapeDtypeStruct((NC, N, D), jnp.float32)]
    if with_cnt:
        out_type.append(jax.ShapeDtypeStruct((NW, 1, N), jnp.float32))
    return pl.kernel(
        _sc_body(with_cnt),
        out_type=tuple(out_type),
        mesh=_mesh,
        scratch_types=(
            [
                pltpu.VMEM((CB, B), jnp.int32),
                pltpu.VMEM((CB, B), jnp.int32),
                pltpu.VMEM((B, D), jnp.float32),
                pltpu.VMEM((B, D), jnp.float32),
                pltpu.VMEM((B, D), jnp.float32),
            ]
            + ([pltpu.VMEM((1, N), jnp.float32)] if with_cnt else [])
            + [
                pltpu.VMEM_SHARED((N, D), jnp.float32),
            ]
            + [pltpu.SemaphoreType.DMA] * 6
        ),
        name="sc_sage_agg_cnt" if with_cnt else "sc_sage_agg",
        compiler_params=pltpu.CompilerParams(needs_layout_passes=False),
    )


_sc_agg_cnt = _make_sc(True)
_sc_agg = _make_sc(False)

def _tc_pre_body(x_ref, wr_ref, b_ref, o_ref):
    o_ref[...] = (
        jnp.dot(x_ref[...], wr_ref[...], preferred_element_type=jnp.float32)
        + b_ref[...]
    )


def _tc_pre(x, wrT, b2d):
    return pl.pallas_call(
        _tc_pre_body,
        out_shape=jax.ShapeDtypeStruct((N, D), jnp.float32),
        name="tc_sage_pre",
    )(x, wrT, b2d)


def _mean_lin(ap_ref, cp_ref, xr_ref, wl_ref):
    p = ap_ref[...]
    agg = p[0] + p[1]
    cnt = jnp.sum(cp_ref[...], axis=(0, 1))
    scale = 1.0 / jnp.maximum(cnt, 1.0)
    m = agg * scale[:, None]
    y = jnp.dot(m, wl_ref[...], preferred_element_type=jnp.float32)
    return y + xr_ref[...]


def _tc_post_body(leaky):
    def body(ap_ref, cp_ref, xr_ref, wl_ref, o_ref):
        y = _mean_lin(ap_ref, cp_ref, xr_ref, wl_ref)
        if leaky:
            y = jnp.where(y > 0, y, 0.01 * y)
        o_ref[...] = y

    return body


def _tc_post(agg_parts, cnt_flat, xr, wlT, leaky):
    return pl.pallas_call(
        _tc_post_body(leaky),
        out_shape=jax.ShapeDtypeStruct((N, D), jnp.float32),
        name="tc_sage_post",
    )(agg_parts, cnt_flat, xr, wlT)


@jax.jit
def kernel(features, edges, edges2, edge_features, additional_feature,
           W1l, W1r, b1, W2l, W2r, b2):
    e5 = edges.astype(jnp.int32).reshape(2, NW, NG, CB, B)

    agg1, cnt3 = _sc_agg_cnt(e5, features)
    xr1 = _tc_pre(features, W1r.T, b1.reshape(1, D))
    h = _tc_post(agg1, cnt3, xr1, W1l.T, leaky=True)
    (agg2,) = _sc_agg(e5, h)
    xr2 = _tc_pre(h, W2r.T, b2.reshape(1, D))
    out = _tc_post(agg2, cnt3, xr2, W2l.T, leaky=False)
    return out
